# baseline (speedup 1.0000x reference)
import functools

import jax
import jax.numpy as jnp
from jax import lax
from jax.experimental import pallas as pl
from jax.experimental.pallas import tpu as pltpu
from jax.experimental.pallas import tpu_sc as plsc

D = 128
N_KW = 50000
N_RT = 10000

K = 128
KC = 128
NTILES = 32

C_KW = 11264
NCH_KW = 5
NPAD_KW = C_KW * NCH_KW
C_RT = 10240
NCH_RT = 1
NPAD_RT = C_RT * NCH_RT

EPAD_RK = 503808
EPAD_KK = 503808
EPAD_RR = 323584


def _vsc_mesh():
    return plsc.VectorSubcoreMesh(
        core_axis_name="c", subcore_axis_name="s", num_cores=2,
        num_subcores=16)


def _segsum(npad, nchunk, csz, epad):
    acc_r = csz + 256
    blocks = epad // (NTILES * K)
    rpt_acc = acc_r // 16
    zgroups = rpt_acc // 16
    rpt_out = csz // 16
    wgroups = (csz // 16) // 64
    assert wgroups * 64 * 16 == csz

    scratch = [
        pltpu.VMEM((K,), jnp.int32),
        pltpu.VMEM((K,), jnp.int32),
        pltpu.VMEM((K,), jnp.int32),
        pltpu.VMEM((16,), jnp.int32),
        pltpu.VMEM((64,), jnp.int32),
        pltpu.VMEM((K, D), jnp.float32),
        pltpu.VMEM((16, D), jnp.float32),
        pltpu.VMEM_SHARED((acc_r, D), jnp.float32),
        pltpu.SemaphoreType.DMA,
    ]

    def body(src_hbm, dst_hbm, table_hbm, z16_hbm, out_hbm,
             idx_s, idx_d, idx_l, idx16, idx64, rows, zrow, acc, sem):
        cid = lax.axis_index("c")
        sid = lax.axis_index("s")
        wid = sid * 2 + cid
        i16 = lax.iota(jnp.int32, 16)

        pltpu.sync_copy(z16_hbm, zrow)

        for c in range(nchunk):
            base_dst = c * csz

            @pl.loop(0, zgroups)
            def _(zb):
                r = sid * rpt_acc + zb * 16
                idx16[...] = i16 + r
                pltpu.sync_copy(zrow, acc.at[idx16])

            plsc.subcore_barrier()

            @pl.loop(0, blocks)
            def _(b):
                ebase = (wid * blocks + b) * K
                pltpu.sync_copy(src_hbm.at[pl.ds(ebase, K)], idx_s)
                pltpu.sync_copy(dst_hbm.at[pl.ds(ebase, K)], idx_d)

                @pl.loop(0, K, step=16)
                def _(i):
                    dv = idx_d[pl.ds(i, 16)]
                    lv = dv - base_dst
                    ok = (lv >= 0) & (lv < csz)
                    idx_l[pl.ds(i, 16)] = jnp.where(ok, lv, csz)

                pltpu.async_copy(table_hbm.at[idx_s], rows, sem).wait()
                pltpu.sync_copy(rows, acc.at[idx_l], add=True)

            plsc.subcore_barrier()

            r0 = sid * rpt_out

            @pl.loop(0, wgroups)
            def _(g):
                start = r0 + g * 64

                @pl.loop(0, 64, step=16)
                def _(j):
                    idx64[pl.ds(j, 16)] = i16 + (start + j)

                pltpu.sync_copy(acc.at[idx64], rows.at[pl.ds(0, 64)])
                pltpu.sync_copy(
                    rows.at[pl.ds(0, 64)],
                    out_hbm.at[cid].at[pl.ds(base_dst + start, 64)])

            plsc.subcore_barrier()

    return pl.kernel(
        body, out_type=jax.ShapeDtypeStruct((2, npad, D), jnp.float32),
        mesh=_vsc_mesh(), scratch_types=scratch)


def _segcnt(npad, outr, epad):
    acc_r = outr + 256
    blocks = epad // (NTILES * K)
    rpt_acc = acc_r // 16
    zgroups = rpt_acc // 16
    assert zgroups * 16 * 16 == acc_r
    rpt_out = outr // 16
    wgroups = rpt_out // 16
    assert wgroups * 16 * 16 == outr
    dump = npad // 8

    scratch = [
        pltpu.VMEM((K,), jnp.int32),
        pltpu.VMEM((K,), jnp.int32),
        pltpu.VMEM((K,), jnp.int32),
        pltpu.VMEM((16,), jnp.int32),
        pltpu.VMEM((K, D), jnp.float32),
        pltpu.VMEM((16, D), jnp.float32),
        pltpu.VMEM_SHARED((acc_r, D), jnp.float32),
        pltpu.SemaphoreType.DMA,
    ]

    def body(dst_hbm, ones8_hbm, z16_hbm, cnt_hbm,
             idx_d, idx_q, idx_m, idx16, rows, zrow, cacc, sem):
        cid = lax.axis_index("c")
        sid = lax.axis_index("s")
        wid = sid * 2 + cid
        i16 = lax.iota(jnp.int32, 16)

        pltpu.sync_copy(z16_hbm, zrow)

        @pl.loop(0, zgroups)
        def _(zb):
            r = sid * rpt_acc + zb * 16
            idx16[...] = i16 + r
            pltpu.sync_copy(zrow, cacc.at[idx16])

        plsc.subcore_barrier()

        @pl.loop(0, blocks)
        def _(b):
            ebase = (wid * blocks + b) * K
            pltpu.sync_copy(dst_hbm.at[pl.ds(ebase, K)], idx_d)

            @pl.loop(0, K, step=16)
            def _(i):
                dv = idx_d[pl.ds(i, 16)]
                idx_q[pl.ds(i, 16)] = jnp.minimum(
                    lax.shift_right_logical(dv, 3), dump)
                idx_m[pl.ds(i, 16)] = dv & 7

            pltpu.async_copy(ones8_hbm.at[idx_m], rows, sem).wait()
            pltpu.sync_copy(rows, cacc.at[idx_q], add=True)

        plsc.subcore_barrier()

        r0 = sid * rpt_out

        @pl.loop(0, wgroups)
        def _(g):
            start = r0 + g * 16
            idx16[...] = i16 + start
            pltpu.sync_copy(cacc.at[idx16], rows.at[pl.ds(0, 16)])
            pltpu.sync_copy(rows.at[pl.ds(0, 16)],
                            cnt_hbm.at[cid].at[pl.ds(start, 16)])

    return pl.kernel(
        body, out_type=jax.ShapeDtypeStruct((2, outr, D), jnp.float32),
        mesh=_vsc_mesh(), scratch_types=scratch)


_ss_kw = _segsum(NPAD_KW, NCH_KW, C_KW, EPAD_RK)
_ss_rt = _segsum(NPAD_RT, NCH_RT, C_RT, EPAD_RR)
OUTR_KW = 7168
OUTR_RT = 1280
_cnt_kw = _segcnt(NPAD_KW, OUTR_KW, EPAD_RK)
_cnt_rt = _segcnt(NPAD_RT, OUTR_RT, EPAD_RR)


def _mm(a, b):
    return lax.dot_general(a, b, (((1,), (0,)), ((), ())),
                           preferred_element_type=jnp.float32)


def _dense2_body(relu, agg_a, cnt_a, agg_b, cnt_b, x,
                 wla, wlb, wra, wrb, ba, bb, out):
    aa = agg_a[...]
    ca = cnt_a[...]
    ab = agg_b[...]
    cb = cnt_b[...]
    xv = x[...]
    mean_a = (aa[0] + aa[1]) / jnp.maximum(ca[0, :, 0:1] + ca[1, :, 0:1], 1.0)
    mean_b = (ab[0] + ab[1]) / jnp.maximum(cb[0, :, 0:1] + cb[1, :, 0:1], 1.0)
    h = (_mm(mean_a, wla[...]) + _mm(mean_b, wlb[...])
         + _mm(xv, wra[...] + wrb[...]) + ba[...] + bb[...])
    out[...] = jnp.maximum(h, 0.0) if relu else h


def _dense1_body(relu, agg, cnt, x, wl, wr, b, out):
    aa = agg[...]
    ca = cnt[...]
    mean = (aa[0] + aa[1]) / jnp.maximum(ca[0, :, 0:1] + ca[1, :, 0:1], 1.0)
    h = _mm(mean, wl[...]) + _mm(x[...], wr[...]) + b[...]
    out[...] = jnp.maximum(h, 0.0) if relu else h


R = 512


def _dense2(npad, relu):
    full = lambda shape: pl.BlockSpec(shape, lambda i: tuple(0 for _ in shape))
    return pl.pallas_call(
        functools.partial(_dense2_body, relu),
        grid=(npad // R,),
        in_specs=[
            pl.BlockSpec((2, R, D), lambda i: (0, i, 0)),
            pl.BlockSpec((2, R, 16), lambda i: (0, i, 0)),
            pl.BlockSpec((2, R, D), lambda i: (0, i, 0)),
            pl.BlockSpec((2, R, 16), lambda i: (0, i, 0)),
            pl.BlockSpec((R, D), lambda i: (i, 0)),
            full((D, D)), full((D, D)), full((D, D)), full((D, D)),
            full((1, D)), full((1, D)),
        ],
        out_specs=pl.BlockSpec((R, D), lambda i: (i, 0)),
        out_shape=jax.ShapeDtypeStruct((npad, D), jnp.float32),
    )


def _dense1(npad, relu):
    full = lambda shape: pl.BlockSpec(shape, lambda i: tuple(0 for _ in shape))
    return pl.pallas_call(
        functools.partial(_dense1_body, relu),
        grid=(npad // R,),
        in_specs=[
            pl.BlockSpec((2, R, D), lambda i: (0, i, 0)),
            pl.BlockSpec((2, R, 16), lambda i: (0, i, 0)),
            pl.BlockSpec((R, D), lambda i: (i, 0)),
            full((D, D)), full((D, D)), full((1, D)),
        ],
        out_specs=pl.BlockSpec((R, D), lambda i: (i, 0)),
        out_shape=jax.ShapeDtypeStruct((npad, D), jnp.float32),
    )


_dense_kw1 = _dense2(NPAD_KW, True)
_dense_kw2 = _dense2(NPAD_KW, False)
_dense_rt1 = _dense1(NPAD_RT, True)
_dense_rt2 = _dense1(NPAD_RT, False)


def _pad_edges(ei, epad, dst_pad):
    e = ei.shape[1]
    src = jnp.concatenate(
        [ei[0].astype(jnp.int32), jnp.zeros((epad - e,), jnp.int32)])
    dst = jnp.concatenate(
        [ei[1].astype(jnp.int32), jnp.full((epad - e,), dst_pad, jnp.int32)])
    return src, dst


def _pad_rows(x, npad):
    n = x.shape[0]
    return jnp.concatenate([x, jnp.zeros((npad - n, D), x.dtype)])


def kernel(x_relation_type, x_keywords,
           edge_index_relates_to, edge_index_linked_to, edge_index_related_to,
           Wl_c1_rk, bl_c1_rk, Wr_c1_rk,
           Wl_c1_kk, bl_c1_kk, Wr_c1_kk,
           Wl_c1_rr, bl_c1_rr, Wr_c1_rr,
           Wl_c2_rk, bl_c2_rk, Wr_c2_rk,
           Wl_c2_kk, bl_c2_kk, Wr_c2_kk,
           Wl_c2_rr, bl_c2_rr, Wr_c2_rr):
    src_rk, dst_rk = _pad_edges(edge_index_relates_to, EPAD_RK, NPAD_KW)
    src_kk, dst_kk = _pad_edges(edge_index_linked_to, EPAD_KK, NPAD_KW)
    src_rr, dst_rr = _pad_edges(edge_index_related_to, EPAD_RR, NPAD_RT)
    xk_p = _pad_rows(x_keywords, NPAD_KW)
    xr_p = _pad_rows(x_relation_type, NPAD_RT)

    z16 = jnp.zeros((16, D), jnp.float32)
    ones8 = (jnp.arange(D, dtype=jnp.int32)[None, :] // 16
             == jnp.arange(8, dtype=jnp.int32)[:, None]).astype(jnp.float32)

    b1_rk = bl_c1_rk.reshape(1, D)
    b1_kk = bl_c1_kk.reshape(1, D)
    b1_rr = bl_c1_rr.reshape(1, D)
    b2_rk = bl_c2_rk.reshape(1, D)
    b2_kk = bl_c2_kk.reshape(1, D)
    b2_rr = bl_c2_rr.reshape(1, D)

    cnt_rk = _cnt_kw(dst_rk, ones8, z16).reshape(2, OUTR_KW * 8, 16)[
        :, :NPAD_KW]
    cnt_kk = _cnt_kw(dst_kk, ones8, z16).reshape(2, OUTR_KW * 8, 16)[
        :, :NPAD_KW]
    cnt_rr = _cnt_rt(dst_rr, ones8, z16).reshape(2, OUTR_RT * 8, 16)

    agg1_rk = _ss_kw(src_rk, dst_rk, x_relation_type, z16)
    agg1_kk = _ss_kw(src_kk, dst_kk, x_keywords, z16)
    agg1_rr = _ss_rt(src_rr, dst_rr, x_relation_type, z16)
    kw1 = _dense_kw1(agg1_rk, cnt_rk, agg1_kk, cnt_kk, xk_p,
                     Wl_c1_rk, Wl_c1_kk, Wr_c1_rk, Wr_c1_kk, b1_rk, b1_kk)
    rt1 = _dense_rt1(agg1_rr, cnt_rr, xr_p, Wl_c1_rr, Wr_c1_rr, b1_rr)

    agg2_rk = _ss_kw(src_rk, dst_rk, rt1, z16)
    agg2_kk = _ss_kw(src_kk, dst_kk, kw1, z16)
    agg2_rr = _ss_rt(src_rr, dst_rr, rt1, z16)
    kw2 = _dense_kw2(agg2_rk, cnt_rk, agg2_kk, cnt_kk, kw1,
                     Wl_c2_rk, Wl_c2_kk, Wr_c2_rk, Wr_c2_kk, b2_rk, b2_kk)
    rt2 = _dense_rt2(agg2_rr, cnt_rr, rt1, Wl_c2_rr, Wr_c2_rr, b2_rr)

    return rt2[:N_RT], kw2[:N_KW]
